# baseline (speedup 1.0000x reference)
import functools

import jax
import jax.numpy as jnp
from jax.experimental import pallas as pl

_ALPHA = 1e-05
_C = 44
_S = 224 * 224
_B = 8
_T = 3584


def _body(f_ref, m_ref, out_ref):
    b = pl.program_id(0)
    t = pl.program_id(1)

    x = f_ref[0]
    f = (jnp.tanh(x) + 1.0) * 0.5
    s = jnp.sum(f, axis=0, keepdims=True)
    fn = f / jnp.maximum(s, 1e-12)
    ent = fn * jnp.log(fn + 1e-4)
    row = jnp.sum(ent, axis=0)

    msel = m_ref[0, 0] == 1
    part_ent = jnp.sum(jnp.where(msel, row, 0.0))
    part_cnt = jnp.sum(msel.astype(jnp.float32))

    lane = jax.lax.broadcasted_iota(jnp.int32, (1, 128), 1)
    v = jnp.where(lane == 0, part_ent, 0.0) + jnp.where(lane == 1, part_cnt, 0.0)

    @pl.when(jnp.logical_and(b == 0, t == 0))
    def _init():
        out_ref[...] = jnp.zeros_like(out_ref)

    out_ref[...] += v


@jax.jit
def kernel(feature, mask):
    f3 = feature.reshape(_B, _C, _S)
    m3 = mask.reshape(_B, 1, _S)

    grid = (_B, _S // _T)
    out = pl.pallas_call(
        _body,
        grid=grid,
        in_specs=[
            pl.BlockSpec((1, _C, _T), lambda b, t: (b, 0, t)),
            pl.BlockSpec((1, 1, _T), lambda b, t: (b, 0, t)),
        ],
        out_specs=pl.BlockSpec((1, 128), lambda b, t: (0, 0)),
        out_shape=jax.ShapeDtypeStruct((1, 128), jnp.float32),
    )(f3, m3)

    ent_sum = -out[0, 0]
    cnt = out[0, 1]
    loss = _ALPHA * ent_sum / (_C * jnp.maximum(cnt, 1.0))
    return jnp.where(cnt == 0.0, jnp.float32(0.0), loss.astype(jnp.float32))


# speedup vs baseline: 2.2705x; 2.2705x over previous
import jax
import jax.numpy as jnp
from jax.experimental import pallas as pl

_ALPHA = 1e-05
_C = 44
_H = 224
_W = 224
_B = 8
_R = 16
_LN2 = 0.6931471805599453


def _body(f_ref, m_ref, out_ref):
    b = pl.program_id(0)
    t = pl.program_id(1)

    g = jnp.tanh(f_ref[0])
    s = jnp.sum(g, axis=0, keepdims=True)
    q = 0.5 / jnp.maximum((s + _C) * 0.5, 1e-12)
    fn = g * q + q
    ent = fn * jnp.log2(fn + 1e-4)
    row = jnp.sum(ent, axis=0)

    msel = m_ref[0] == 1
    part_ent = jnp.sum(jnp.where(msel, row, 0.0))
    part_cnt = jnp.sum(msel.astype(jnp.float32))

    lane = jax.lax.broadcasted_iota(jnp.int32, (1, 128), 1)
    v = jnp.where(lane == 0, part_ent, 0.0) + jnp.where(lane == 1, part_cnt, 0.0)

    @pl.when(jnp.logical_and(b == 0, t == 0))
    def _init():
        out_ref[...] = jnp.zeros_like(out_ref)

    out_ref[...] += v


@jax.jit
def kernel(feature, mask):
    grid = (_B, _H // _R)
    out = pl.pallas_call(
        _body,
        grid=grid,
        in_specs=[
            pl.BlockSpec((1, _C, _R, _W), lambda b, t: (b, 0, t, 0)),
            pl.BlockSpec((1, _R, _W), lambda b, t: (b, t, 0)),
        ],
        out_specs=pl.BlockSpec((1, 128), lambda b, t: (0, 0)),
        out_shape=jax.ShapeDtypeStruct((1, 128), jnp.float32),
    )(feature, mask)

    ent_sum = -out[0, 0] * _LN2
    cnt = out[0, 1]
    loss = _ALPHA * ent_sum / (_C * jnp.maximum(cnt, 1.0))
    return jnp.where(cnt == 0.0, jnp.float32(0.0), loss.astype(jnp.float32))


# speedup vs baseline: 3.5318x; 1.5555x over previous
import jax
import jax.numpy as jnp
from jax.experimental import pallas as pl

_ALPHA = 1e-05
_C = 44
_H = 224
_W = 224
_B = 8
_R = 56
_LN2 = 0.6931471805599453


def _body(f_ref, m_ref, out_ref):
    b = pl.program_id(0)
    t = pl.program_id(1)

    g = jnp.tanh(f_ref[0])
    s = jnp.sum(g, axis=0, keepdims=True)
    q = 0.5 / jnp.maximum((s + _C) * 0.5, 1e-12)
    fn = g * q + q
    ent = fn * jnp.log2(fn + 1e-4)
    row = jnp.sum(ent, axis=0)

    msel = m_ref[0] == 1
    part_ent = jnp.sum(jnp.where(msel, row, 0.0))
    part_cnt = jnp.sum(msel.astype(jnp.float32))

    lane = jax.lax.broadcasted_iota(jnp.int32, (1, 128), 1)
    v = jnp.where(lane == 0, part_ent, 0.0) + jnp.where(lane == 1, part_cnt, 0.0)

    @pl.when(jnp.logical_and(b == 0, t == 0))
    def _init():
        out_ref[...] = jnp.zeros_like(out_ref)

    out_ref[...] += v


@jax.jit
def kernel(feature, mask):
    grid = (_B, _H // _R)
    out = pl.pallas_call(
        _body,
        grid=grid,
        in_specs=[
            pl.BlockSpec((1, _C, _R, _W), lambda b, t: (b, 0, t, 0)),
            pl.BlockSpec((1, _R, _W), lambda b, t: (b, t, 0)),
        ],
        out_specs=pl.BlockSpec((1, 128), lambda b, t: (0, 0)),
        out_shape=jax.ShapeDtypeStruct((1, 128), jnp.float32),
    )(feature, mask)

    ent_sum = -out[0, 0] * _LN2
    cnt = out[0, 1]
    loss = _ALPHA * ent_sum / (_C * jnp.maximum(cnt, 1.0))
    return jnp.where(cnt == 0.0, jnp.float32(0.0), loss.astype(jnp.float32))
